# baseline (speedup 1.0000x reference)
import functools

import jax
import jax.numpy as jnp
import numpy as np
from jax.experimental import pallas as pl

_NSAMPLE = 32
_NPTS = (512, 128, 32, 8)
_RADII = (0.1, 0.2, 0.4, 0.8)
_INTERPRET = False


def _fps_body(x_ref, y_ref, z_ref, ox_ref, oy_ref, oz_ref, *, npoint):
    b, n = x_ref.shape
    x = x_ref[...]
    y = y_ref[...]
    z = z_ref[...]
    iota_n = jax.lax.broadcasted_iota(jnp.int32, (b, n), 1)
    iota_p = jax.lax.broadcasted_iota(jnp.int32, (b, npoint), 1)

    def body(i, state):
        dist, far, ox, oy, oz = state
        sel = iota_n == far
        cx = jnp.sum(jnp.where(sel, x, 0.0), axis=1, keepdims=True)
        cy = jnp.sum(jnp.where(sel, y, 0.0), axis=1, keepdims=True)
        cz = jnp.sum(jnp.where(sel, z, 0.0), axis=1, keepdims=True)
        hit = iota_p == i
        ox = jnp.where(hit, cx, ox)
        oy = jnp.where(hit, cy, oy)
        oz = jnp.where(hit, cz, oz)
        tx = x - cx
        ty = y - cy
        tz = z - cz
        d = (tx * tx + ty * ty) + tz * tz
        dist = jnp.minimum(dist, d)
        m = jnp.max(dist, axis=1, keepdims=True)
        far = jnp.min(jnp.where(dist == m, iota_n, n), axis=1, keepdims=True)
        return dist, far, ox, oy, oz

    init = (jnp.full((b, n), 1e10, jnp.float32),
            jnp.zeros((b, 1), jnp.int32),
            jnp.zeros((b, npoint), jnp.float32),
            jnp.zeros((b, npoint), jnp.float32),
            jnp.zeros((b, npoint), jnp.float32))
    _, _, ox, oy, oz = jax.lax.fori_loop(0, npoint, body, init)
    ox_ref[...] = ox
    oy_ref[...] = oy
    oz_ref[...] = oz


def _fps(xyz, npoint):
    b = xyz.shape[0]
    outs = pl.pallas_call(
        functools.partial(_fps_body, npoint=npoint),
        out_shape=[jax.ShapeDtypeStruct((b, npoint), jnp.float32)] * 3,
        interpret=_INTERPRET,
    )(xyz[:, :, 0], xyz[:, :, 1], xyz[:, :, 2])
    return jnp.stack(outs, axis=2)


def _sa_body(*refs, n, sb, nsample, r2, nw):
    cxyz_ref, xyz_ref, table_ref = refs[0], refs[1], refs[2]
    wrefs = refs[3:3 + nw]
    out_ref = refs[3 + nw]

    cx = cxyz_ref[0]
    xx = xyz_ref[0]
    t = cx[:, 0:1] - xx[0:1, :]
    d = t * t
    t = cx[:, 1:2] - xx[1:2, :]
    d = d + t * t
    t = cx[:, 2:3] - xx[2:3, :]
    d = d + t * t

    iota = jax.lax.broadcasted_iota(jnp.int32, (sb, n), 1)
    val = jnp.where(d > r2, n, iota)
    big = np.int32(2 ** 30)
    cols = []
    for _ in range(nsample):
        mk = jnp.min(val, axis=1, keepdims=True)
        cols.append(mk)
        val = jnp.where(val == mk, big, val)
    first = cols[0]

    table = table_ref[0]
    rows = []
    for k in range(nsample):
        gk = jnp.where(cols[k] >= n, first, cols[k])
        oh = (iota == gk).astype(jnp.float32)
        g = jnp.dot(oh, table, preferred_element_type=jnp.float32)
        g = jnp.concatenate([g[:, 0:3] - cx, g[:, 3:]], axis=1)
        rows.append(g)
    x = jnp.concatenate(rows, axis=0)

    for j in range(nw):
        w = wrefs[j][...]
        x = jnp.maximum(jnp.dot(x, w, preferred_element_type=jnp.float32), 0.0)
    cout = x.shape[1]
    x = x.reshape(nsample, sb, cout)
    out_ref[0] = jnp.max(x, axis=0)


def _sa_level(xyz, points, npoint, radius, ws):
    b, n, _ = xyz.shape
    xyz_t = jnp.transpose(xyz, (0, 2, 1))
    new_xyz = _fps(xyz, npoint)
    table = jnp.concatenate([xyz, points], axis=2)
    cin = table.shape[2]
    sb = min(npoint, 256)
    gs = npoint // sb
    cout = ws[-1].shape[1]
    nw = len(ws)
    body = functools.partial(_sa_body, n=n, sb=sb, nsample=_NSAMPLE,
                             r2=np.float32(radius ** 2), nw=nw)
    new_points = pl.pallas_call(
        body,
        grid=(b, gs),
        in_specs=[
            pl.BlockSpec((1, sb, 3), lambda i, j: (i, j, 0)),
            pl.BlockSpec((1, 3, n), lambda i, j: (i, 0, 0)),
            pl.BlockSpec((1, n, cin), lambda i, j: (i, 0, 0)),
        ] + [pl.BlockSpec(w.shape, lambda i, j: (0, 0)) for w in ws],
        out_specs=pl.BlockSpec((1, sb, cout), lambda i, j: (i, j, 0)),
        out_shape=jax.ShapeDtypeStruct((b, npoint, cout), jnp.float32),
        interpret=_INTERPRET,
    )(new_xyz, xyz_t, table, *ws)
    return new_xyz, new_points


def _fp_body(*refs, ns, tb, nw):
    xyzt_ref, xyzs_ref, ft_ref, fs_ref = refs[:4]
    wrefs = refs[4:4 + nw]
    out_ref = refs[4 + nw]

    cx = xyzt_ref[0]
    sx = xyzs_ref[0]
    t = cx[:, 0:1] - sx[0:1, :]
    d = t * t
    t = cx[:, 1:2] - sx[1:2, :]
    d = d + t * t
    t = cx[:, 2:3] - sx[2:3, :]
    d = d + t * t
    dis = jnp.sqrt(jnp.maximum(d, 1e-12))

    iota = jax.lax.broadcasted_iota(jnp.int32, (tb, ns), 1)
    val = dis
    invs, sels = [], []
    for _ in range(3):
        m = jnp.min(val, axis=1, keepdims=True)
        ik = jnp.min(jnp.where(val == m, iota, ns), axis=1, keepdims=True)
        invs.append(1.0 / jnp.maximum(m, 1e-10))
        sels.append(ik)
        val = jnp.where(iota == ik, 1e30, val)
    norm = (invs[0] + invs[1]) + invs[2]
    wks = [inv / norm for inv in invs]
    wsum = ((wks[0] + wks[1]) + wks[2]) + 1e-6
    wmat = (wks[0] * (iota == sels[0]).astype(jnp.float32)
            + wks[1] * (iota == sels[1]).astype(jnp.float32)
            + wks[2] * (iota == sels[2]).astype(jnp.float32))

    fs = fs_ref[0]
    interp = jnp.dot(wmat, fs, preferred_element_type=jnp.float32) / wsum
    x = jnp.concatenate([interp, ft_ref[0]], axis=1)
    for j in range(nw):
        w = wrefs[j][...]
        x = jnp.maximum(jnp.dot(x, w, preferred_element_type=jnp.float32), 0.0)
    out_ref[0] = x


def _fp_level(xyz_target, xyz_source, feats_target, feats_source, ws):
    b, nt, _ = xyz_target.shape
    ns = xyz_source.shape[1]
    xyz_src_t = jnp.transpose(xyz_source, (0, 2, 1))
    cs = feats_source.shape[2]
    ct = feats_target.shape[2]
    tb = min(nt, 512)
    gs = nt // tb
    cout = ws[-1].shape[1]
    nw = len(ws)
    body = functools.partial(_fp_body, ns=ns, tb=tb, nw=nw)
    return pl.pallas_call(
        body,
        grid=(b, gs),
        in_specs=[
            pl.BlockSpec((1, tb, 3), lambda i, j: (i, j, 0)),
            pl.BlockSpec((1, 3, ns), lambda i, j: (i, 0, 0)),
            pl.BlockSpec((1, tb, ct), lambda i, j: (i, j, 0)),
            pl.BlockSpec((1, ns, cs), lambda i, j: (i, 0, 0)),
        ] + [pl.BlockSpec(w.shape, lambda i, j: (0, 0)) for w in ws],
        out_specs=pl.BlockSpec((1, tb, cout), lambda i, j: (i, j, 0)),
        out_shape=jax.ShapeDtypeStruct((b, nt, cout), jnp.float32),
        interpret=_INTERPRET,
    )(xyz_target, xyz_src_t, feats_target, feats_source, *ws)


def kernel(l0_xyz, l0_points, sa_weights, fp_weights):
    xyzs = [l0_xyz]
    feats = [l0_points]
    for i in range(4):
        nx, nf = _sa_level(xyzs[-1], feats[-1], _NPTS[i], _RADII[i],
                           sa_weights[i])
        xyzs.append(nx)
        feats.append(nf)
    l3 = _fp_level(xyzs[3], xyzs[4], feats[3], feats[4], fp_weights[0])
    l2 = _fp_level(xyzs[2], xyzs[3], feats[2], l3, fp_weights[1])
    l1 = _fp_level(xyzs[1], xyzs[2], feats[1], l2, fp_weights[2])
    l0 = _fp_level(xyzs[0], xyzs[1], feats[0], l1, fp_weights[3])
    return l0


# speedup vs baseline: 1.1914x; 1.1914x over previous
import functools

import jax
import jax.numpy as jnp
import numpy as np
from jax.experimental import pallas as pl
from jax.experimental.pallas import tpu as pltpu
from jax.experimental.pallas import tpu_sc as plsc

_NSAMPLE = 32
_NPTS = (512, 128, 32, 8)
_RADII = (0.1, 0.2, 0.4, 0.8)
_INTERPRET = False


def _fps_body(x_ref, y_ref, z_ref, ox_ref, oy_ref, oz_ref, *, npoint):
    b, n = x_ref.shape
    x = x_ref[...]
    y = y_ref[...]
    z = z_ref[...]
    iota_n = jax.lax.broadcasted_iota(jnp.int32, (b, n), 1)
    iota_p = jax.lax.broadcasted_iota(jnp.int32, (b, npoint), 1)

    def body(i, state):
        dist, far, ox, oy, oz = state
        sel = iota_n == far
        cx = jnp.sum(jnp.where(sel, x, 0.0), axis=1, keepdims=True)
        cy = jnp.sum(jnp.where(sel, y, 0.0), axis=1, keepdims=True)
        cz = jnp.sum(jnp.where(sel, z, 0.0), axis=1, keepdims=True)
        hit = iota_p == i
        ox = jnp.where(hit, cx, ox)
        oy = jnp.where(hit, cy, oy)
        oz = jnp.where(hit, cz, oz)
        tx = x - cx
        ty = y - cy
        tz = z - cz
        d = (tx * tx + ty * ty) + tz * tz
        dist = jnp.minimum(dist, d)
        m = jnp.max(dist, axis=1, keepdims=True)
        far = jnp.min(jnp.where(dist == m, iota_n, n), axis=1, keepdims=True)
        return dist, far, ox, oy, oz

    init = (jnp.full((b, n), 1e10, jnp.float32),
            jnp.zeros((b, 1), jnp.int32),
            jnp.zeros((b, npoint), jnp.float32),
            jnp.zeros((b, npoint), jnp.float32),
            jnp.zeros((b, npoint), jnp.float32))
    _, _, ox, oy, oz = jax.lax.fori_loop(0, npoint, body, init)
    ox_ref[...] = ox
    oy_ref[...] = oy
    oz_ref[...] = oz


def _fps(xyz, npoint):
    b = xyz.shape[0]
    outs = pl.pallas_call(
        functools.partial(_fps_body, npoint=npoint),
        out_shape=[jax.ShapeDtypeStruct((b, npoint), jnp.float32)] * 3,
        interpret=_INTERPRET,
    )(xyz[:, :, 0], xyz[:, :, 1], xyz[:, :, 2])
    return jnp.stack(outs, axis=2)


def _sa_body(*refs, n, sb, nsample, r2, nw):
    cxyz_ref, xyz_ref, table_ref = refs[0], refs[1], refs[2]
    wrefs = refs[3:3 + nw]
    out_ref = refs[3 + nw]

    cx = cxyz_ref[0]
    xx = xyz_ref[0]
    t = cx[:, 0:1] - xx[0:1, :]
    d = t * t
    t = cx[:, 1:2] - xx[1:2, :]
    d = d + t * t
    t = cx[:, 2:3] - xx[2:3, :]
    d = d + t * t

    iota = jax.lax.broadcasted_iota(jnp.int32, (sb, n), 1)
    val = jnp.where(d > r2, n, iota)
    big = np.int32(2 ** 30)
    cols = []
    for _ in range(nsample):
        mk = jnp.min(val, axis=1, keepdims=True)
        cols.append(mk)
        val = jnp.where(val == mk, big, val)
    first = cols[0]

    table = table_ref[0]
    rows = []
    for k in range(nsample):
        gk = jnp.where(cols[k] >= n, first, cols[k])
        oh = (iota == gk).astype(jnp.float32)
        g = jnp.dot(oh, table, preferred_element_type=jnp.float32)
        g = jnp.concatenate([g[:, 0:3] - cx, g[:, 3:]], axis=1)
        rows.append(g)
    x = jnp.concatenate(rows, axis=0)

    for j in range(nw):
        w = wrefs[j][...]
        x = jnp.maximum(jnp.dot(x, w, preferred_element_type=jnp.float32), 0.0)
    cout = x.shape[1]
    x = x.reshape(nsample, sb, cout)
    out_ref[0] = jnp.max(x, axis=0)


def _sa_level(xyz, points, npoint, radius, ws):
    b, n, _ = xyz.shape
    xyz_t = jnp.transpose(xyz, (0, 2, 1))
    new_xyz = _fps(xyz, npoint)
    table = jnp.concatenate([xyz, points], axis=2)
    cin = table.shape[2]
    sb = min(npoint, 256)
    gs = npoint // sb
    cout = ws[-1].shape[1]
    nw = len(ws)
    body = functools.partial(_sa_body, n=n, sb=sb, nsample=_NSAMPLE,
                             r2=np.float32(radius ** 2), nw=nw)
    new_points = pl.pallas_call(
        body,
        grid=(b, gs),
        in_specs=[
            pl.BlockSpec((1, sb, 3), lambda i, j: (i, j, 0)),
            pl.BlockSpec((1, 3, n), lambda i, j: (i, 0, 0)),
            pl.BlockSpec((1, n, cin), lambda i, j: (i, 0, 0)),
        ] + [pl.BlockSpec(w.shape, lambda i, j: (0, 0)) for w in ws],
        out_specs=pl.BlockSpec((1, sb, cout), lambda i, j: (i, j, 0)),
        out_shape=jax.ShapeDtypeStruct((b, npoint, cout), jnp.float32),
        interpret=_INTERPRET,
    )(new_xyz, xyz_t, table, *ws)
    return new_xyz, new_points


def _bq_body(cxyz_ref, xyz_ref, out_ref, *, n, sb, nsample, r2):
    cx = cxyz_ref[0]
    xx = xyz_ref[0]
    t = cx[:, 0:1] - xx[0:1, :]
    d = t * t
    t = cx[:, 1:2] - xx[1:2, :]
    d = d + t * t
    t = cx[:, 2:3] - xx[2:3, :]
    d = d + t * t
    iota = jax.lax.broadcasted_iota(jnp.int32, (sb, n), 1)
    val = jnp.where(d > r2, n, iota)
    big = np.int32(2 ** 30)
    cols = []
    for _ in range(nsample):
        mk = jnp.min(val, axis=1, keepdims=True)
        cols.append(mk)
        val = jnp.where(val == mk, big, val)
    first = cols[0]
    grp = jnp.concatenate(
        [jnp.where(c >= n, first, c) for c in cols], axis=1)
    out_ref[0] = grp + pl.program_id(0) * n


def _sc_gather(table, idx, dp):
    info = plsc.get_sparse_core_info()
    nw = info.num_cores * info.num_subcores
    m = idx.shape[0]
    bpw = m // nw
    mesh = plsc.VectorSubcoreMesh(core_axis_name="c", subcore_axis_name="s")

    @functools.partial(
        pl.kernel, mesh=mesh,
        out_type=jax.ShapeDtypeStruct((m, dp), jnp.float32),
        scratch_types=[pltpu.VMEM((bpw,), jnp.int32),
                       pltpu.VMEM((bpw, dp), jnp.float32),
                       pltpu.SemaphoreType.DMA],
        compiler_params=pltpu.CompilerParams(use_tc_tiling_on_sc=False),
    )
    def gk(table_hbm, idx_hbm, out_hbm, idx_v, rows_v, sem):
        wid = jax.lax.axis_index("s") * info.num_cores + jax.lax.axis_index("c")
        base = wid * bpw
        pltpu.sync_copy(idx_hbm.at[pl.ds(base, bpw)], idx_v)
        pltpu.async_copy(table_hbm.at[idx_v], rows_v, sem).wait()
        pltpu.sync_copy(rows_v, out_hbm.at[pl.ds(base, bpw)])

    return gk(table, idx)


def _sa_mlp_body(*refs, sb, nsample, cin, nw):
    g_ref, cxyz_ref = refs[0], refs[1]
    wrefs = refs[2:2 + nw]
    out_ref = refs[2 + nw]
    g = g_ref[0]
    cx = cxyz_ref[0]
    gx = g[:, :, 0:3] - cx[:, None, :]
    x = jnp.concatenate([gx, g[:, :, 3:cin]], axis=2)
    x = x.reshape(sb * nsample, cin)
    for j in range(nw):
        w = wrefs[j][...]
        x = jnp.maximum(jnp.dot(x, w, preferred_element_type=jnp.float32), 0.0)
    cout = x.shape[1]
    x = x.reshape(sb, nsample, cout)
    out_ref[0] = jnp.max(x, axis=1)


def _sa_level_sc(xyz, points, npoint, radius, ws):
    b, n, _ = xyz.shape
    xyz_t = jnp.transpose(xyz, (0, 2, 1))
    new_xyz = _fps(xyz, npoint)
    table = jnp.concatenate([xyz, points], axis=2)
    cin = table.shape[2]
    dp = ((cin + 15) // 16) * 16
    sb = min(npoint, 256)
    gs = npoint // sb
    k = _NSAMPLE
    grp = pl.pallas_call(
        functools.partial(_bq_body, n=n, sb=sb, nsample=k,
                          r2=np.float32(radius ** 2)),
        grid=(b, gs),
        in_specs=[
            pl.BlockSpec((1, sb, 3), lambda i, j: (i, j, 0)),
            pl.BlockSpec((1, 3, n), lambda i, j: (i, 0, 0)),
        ],
        out_specs=pl.BlockSpec((1, sb, k), lambda i, j: (i, j, 0)),
        out_shape=jax.ShapeDtypeStruct((b, npoint, k), jnp.int32),
        interpret=_INTERPRET,
    )(new_xyz, xyz_t)
    table_flat = jnp.pad(table, ((0, 0), (0, 0), (0, dp - cin)))
    table_flat = table_flat.reshape(b * n, dp)
    rows = _sc_gather(table_flat, grp.reshape(-1), dp)
    g4 = rows.reshape(b, npoint, k, dp)
    cout = ws[-1].shape[1]
    nw = len(ws)
    new_points = pl.pallas_call(
        functools.partial(_sa_mlp_body, sb=sb, nsample=k, cin=cin, nw=nw),
        grid=(b, gs),
        in_specs=[
            pl.BlockSpec((1, sb, k, dp), lambda i, j: (i, j, 0, 0)),
            pl.BlockSpec((1, sb, 3), lambda i, j: (i, j, 0)),
        ] + [pl.BlockSpec(w.shape, lambda i, j: (0, 0)) for w in ws],
        out_specs=pl.BlockSpec((1, sb, cout), lambda i, j: (i, j, 0)),
        out_shape=jax.ShapeDtypeStruct((b, npoint, cout), jnp.float32),
        interpret=_INTERPRET,
    )(g4, new_xyz, *ws)
    return new_xyz, new_points


def _fp_body(*refs, ns, tb, nw):
    xyzt_ref, xyzs_ref, ft_ref, fs_ref = refs[:4]
    wrefs = refs[4:4 + nw]
    out_ref = refs[4 + nw]

    cx = xyzt_ref[0]
    sx = xyzs_ref[0]
    t = cx[:, 0:1] - sx[0:1, :]
    d = t * t
    t = cx[:, 1:2] - sx[1:2, :]
    d = d + t * t
    t = cx[:, 2:3] - sx[2:3, :]
    d = d + t * t
    dis = jnp.sqrt(jnp.maximum(d, 1e-12))

    iota = jax.lax.broadcasted_iota(jnp.int32, (tb, ns), 1)
    val = dis
    invs, sels = [], []
    for _ in range(3):
        m = jnp.min(val, axis=1, keepdims=True)
        ik = jnp.min(jnp.where(val == m, iota, ns), axis=1, keepdims=True)
        invs.append(1.0 / jnp.maximum(m, 1e-10))
        sels.append(ik)
        val = jnp.where(iota == ik, 1e30, val)
    norm = (invs[0] + invs[1]) + invs[2]
    wks = [inv / norm for inv in invs]
    wsum = ((wks[0] + wks[1]) + wks[2]) + 1e-6
    wmat = (wks[0] * (iota == sels[0]).astype(jnp.float32)
            + wks[1] * (iota == sels[1]).astype(jnp.float32)
            + wks[2] * (iota == sels[2]).astype(jnp.float32))

    fs = fs_ref[0]
    interp = jnp.dot(wmat, fs, preferred_element_type=jnp.float32) / wsum
    x = jnp.concatenate([interp, ft_ref[0]], axis=1)
    for j in range(nw):
        w = wrefs[j][...]
        x = jnp.maximum(jnp.dot(x, w, preferred_element_type=jnp.float32), 0.0)
    out_ref[0] = x


def _fp_level(xyz_target, xyz_source, feats_target, feats_source, ws):
    b, nt, _ = xyz_target.shape
    ns = xyz_source.shape[1]
    xyz_src_t = jnp.transpose(xyz_source, (0, 2, 1))
    cs = feats_source.shape[2]
    ct = feats_target.shape[2]
    tb = min(nt, 512)
    gs = nt // tb
    cout = ws[-1].shape[1]
    nw = len(ws)
    body = functools.partial(_fp_body, ns=ns, tb=tb, nw=nw)
    return pl.pallas_call(
        body,
        grid=(b, gs),
        in_specs=[
            pl.BlockSpec((1, tb, 3), lambda i, j: (i, j, 0)),
            pl.BlockSpec((1, 3, ns), lambda i, j: (i, 0, 0)),
            pl.BlockSpec((1, tb, ct), lambda i, j: (i, j, 0)),
            pl.BlockSpec((1, ns, cs), lambda i, j: (i, 0, 0)),
        ] + [pl.BlockSpec(w.shape, lambda i, j: (0, 0)) for w in ws],
        out_specs=pl.BlockSpec((1, tb, cout), lambda i, j: (i, j, 0)),
        out_shape=jax.ShapeDtypeStruct((b, nt, cout), jnp.float32),
        interpret=_INTERPRET,
    )(xyz_target, xyz_src_t, feats_target, feats_source, *ws)


def kernel(l0_xyz, l0_points, sa_weights, fp_weights):
    xyzs = [l0_xyz]
    feats = [l0_points]
    for i in range(4):
        sa_fn = _sa_level_sc if i == 0 else _sa_level
        nx, nf = sa_fn(xyzs[-1], feats[-1], _NPTS[i], _RADII[i],
                       sa_weights[i])
        xyzs.append(nx)
        feats.append(nf)
    l3 = _fp_level(xyzs[3], xyzs[4], feats[3], feats[4], fp_weights[0])
    l2 = _fp_level(xyzs[2], xyzs[3], feats[2], l3, fp_weights[1])
    l1 = _fp_level(xyzs[1], xyzs[2], feats[1], l2, fp_weights[2])
    l0 = _fp_level(xyzs[0], xyzs[1], feats[0], l1, fp_weights[3])
    return l0
